# baseline (speedup 1.0000x reference)
import jax
import jax.numpy as jnp
from jax import lax
from jax.experimental import pallas as pl
from jax.experimental.pallas import tpu as pltpu, tpu_sc as plsc

_D = 768
_B = 32
_NC, _NS = 2, 16
_NW = _NC * _NS


def _sc_body(x_hbm, idx_hbm, pe_hbm, out_hbm, pe_v, xb, ibv, sem):
    c = lax.axis_index("c")
    s = lax.axis_index("s")
    wid = s * _NC + c

    pltpu.sync_copy(pe_hbm, pe_v)

    rows_per_w = x_hbm.shape[0] // _NW
    chunks = rows_per_w // _B

    def chunk(g, carry):
        base = wid * rows_per_w + g * _B
        pltpu.sync_copy(x_hbm.at[pl.ds(base, _B)], xb)
        pltpu.sync_copy(idx_hbm.at[pl.ds(base, _B)], ibv)

        def group(k, carry2):
            iv16 = ibv[pl.ds(16 * k, 16)]
            for l in range(16):
                ds = iv16[l]
                row = 16 * k + l

                @plsc.parallel_loop(0, _D // 16, unroll=8)
                def _(j):
                    off = 16 * j
                    plsc.addupdate(xb.at[row, pl.ds(off, 16)],
                                   pe_v[ds, pl.ds(off, 16)])
            return carry2

        lax.fori_loop(0, _B // 16, group, 0)
        pltpu.sync_copy(xb, out_hbm.at[pl.ds(base, _B)])
        return carry

    lax.fori_loop(0, chunks, chunk, 0)


def kernel(x, layer_index, pe):
    n = x.shape[0]
    pe2 = pe.reshape(pe.shape[0], _D)
    k = pl.kernel(
        _sc_body,
        out_type=jax.ShapeDtypeStruct((n, _D), jnp.float32),
        mesh=plsc.VectorSubcoreMesh(core_axis_name="c", subcore_axis_name="s",
                                    num_cores=_NC, num_subcores=_NS),
        scratch_types=[
            pltpu.VMEM((100, _D), jnp.float32),
            pltpu.VMEM((_B, _D), jnp.float32),
            pltpu.VMEM((_B,), jnp.int32),
            pltpu.SemaphoreType.DMA,
        ],
    )
    return k(x, layer_index, pe2)


# speedup vs baseline: 1.5285x; 1.5285x over previous
import jax
import jax.numpy as jnp
from jax import lax
from jax.experimental import pallas as pl
from jax.experimental.pallas import tpu as pltpu, tpu_sc as plsc

_D = 768
_B = 16
_NSLOT = 4
_NC, _NS = 2, 16
_NW = _NC * _NS


def _sc_body(x_hbm, idx_hbm, pe_hbm, out_hbm, pe_v, *rest):
    xbs = rest[0:4]
    ibs = rest[4:8]
    lxs = rest[8:12]
    lis = rest[12:16]
    sts = rest[16:20]

    c = lax.axis_index("c")
    s = lax.axis_index("s")
    wid = s * _NC + c
    rows_per_w = x_hbm.shape[0] // _NW
    chunks = rows_per_w // _B
    base0 = wid * rows_per_w

    pltpu.sync_copy(pe_hbm, pe_v)

    def start_load(g, t):
        b = base0 + g * _B
        pltpu.async_copy(x_hbm.at[pl.ds(b, _B)], xbs[t], lxs[t])
        pltpu.async_copy(idx_hbm.at[pl.ds(b, _B)], ibs[t], lis[t])

    def wait_load(t):
        pltpu.make_async_copy(x_hbm.at[pl.ds(0, _B)], xbs[t], lxs[t]).wait()
        pltpu.make_async_copy(idx_hbm.at[pl.ds(0, _B)], ibs[t], lis[t]).wait()

    def start_store(g, t):
        b = base0 + g * _B
        pltpu.async_copy(xbs[t], out_hbm.at[pl.ds(b, _B)], sts[t])

    def wait_store(t):
        pltpu.make_async_copy(xbs[t], out_hbm.at[pl.ds(0, _B)], sts[t]).wait()

    def compute(t):
        iv16 = ibs[t][...]
        xb = xbs[t]
        for l in range(_B):
            ds = iv16[l]

            @plsc.parallel_loop(0, _D // 16, unroll=8)
            def _(j):
                off = 16 * j
                plsc.addupdate(xb.at[l, pl.ds(off, 16)],
                               pe_v[ds, pl.ds(off, 16)])

    def step(g, t, reload_ok, first_round):
        wait_load(t)
        compute(t)
        start_store(g, t)
        if reload_ok:
            nt = (t + 3) % _NSLOT
            if not first_round:
                wait_store(nt)
            start_load(g + 3, nt)

    for t in range(3):
        start_load(t, t)
    for g in range(4):
        step(g, g % _NSLOT, True, g == 0)

    def body(h, carry):
        g0 = 4 * h
        for t in range(_NSLOT):
            step(g0 + t, t, True, False)
        return carry

    lax.fori_loop(1, chunks // 4 - 1, body, 0)

    gt = chunks - 4
    for t in range(_NSLOT):
        g = gt + t
        step(g, t, g + 3 < chunks, False)
    for t in range(_NSLOT):
        wait_store(t)


def kernel(x, layer_index, pe):
    n = x.shape[0]
    pe2 = pe.reshape(pe.shape[0], _D)
    k = pl.kernel(
        _sc_body,
        out_type=jax.ShapeDtypeStruct((n, _D), jnp.float32),
        mesh=plsc.VectorSubcoreMesh(core_axis_name="c", subcore_axis_name="s",
                                    num_cores=_NC, num_subcores=_NS),
        scratch_types=(
            [pltpu.VMEM((100, _D), jnp.float32)]
            + [pltpu.VMEM((_B, _D), jnp.float32) for _ in range(_NSLOT)]
            + [pltpu.VMEM((_B,), jnp.int32) for _ in range(_NSLOT)]
            + [pltpu.SemaphoreType.DMA for _ in range(3 * _NSLOT)]
        ),
    )
    return k(x, layer_index, pe2)
